# baseline (speedup 1.0000x reference)
import functools

import jax
import jax.numpy as jnp
from jax import lax
from jax.experimental import pallas as pl
from jax.experimental.pallas import tpu as pltpu
from jax.experimental.pallas import tpu_sc as plsc

BATCH = 16384
HIST = 20
EMBED_DIM = 16
NC = 2
NS = 16
NW = NC * NS
VPW = BATCH // NW
IDX_MINOR = 128
IDX_ROWS = VPW * HIST // IDX_MINOR
CHUNK_V = 128
CHUNK_ROWS = CHUNK_V * HIST
CHUNK_DMAS = CHUNK_ROWS // IDX_MINOR
N_CHUNKS = VPW // CHUNK_V

_mesh = plsc.VectorSubcoreMesh(core_axis_name="c", subcore_axis_name="s")

_GATHER_DNUMS = lax.GatherDimensionNumbers(
    offset_dims=(), collapsed_slice_dims=(0,), start_index_map=(0,))


def _lane_gather(x, idx):
    return lax.gather(
        x, idx[:, None], _GATHER_DNUMS, (1,),
        mode=lax.GatherScatterMode.PROMISE_IN_BOUNDS)


def _tree_sum(vals):
    while len(vals) > 1:
        nxt = [a + b for a, b in zip(vals[::2], vals[1::2])]
        if len(vals) % 2:
            nxt.append(vals[-1])
        vals = nxt
    return vals[0]


@functools.partial(
    pl.kernel,
    out_type=jax.ShapeDtypeStruct((BATCH, EMBED_DIM), jnp.float32),
    mesh=_mesh,
    scratch_types=[
        pltpu.VMEM((IDX_ROWS, IDX_MINOR), jnp.int32),
        pltpu.VMEM((CHUNK_ROWS, EMBED_DIM), jnp.float32),
        pltpu.VMEM((CHUNK_ROWS, EMBED_DIM), jnp.float32),
        pltpu.VMEM((VPW, EMBED_DIM), jnp.float32),
        pltpu.SemaphoreType.DMA,
        pltpu.SemaphoreType.DMA,
    ],
    compiler_params=pltpu.CompilerParams(use_tc_tiling_on_sc=False),
)
def _sc_encode(idx_hbm, table_hbm, out_hbm, idx_v, rows_a, rows_b, out_v,
               sem_a, sem_b):
    wid = lax.axis_index("s") * NC + lax.axis_index("c")
    pltpu.sync_copy(idx_hbm.at[wid], idx_v)

    bufs = (rows_a, rows_b)
    sems = (sem_a, sem_b)

    def fire(c):
        buf, sem = bufs[c % 2], sems[c % 2]
        cps = []
        for j in range(CHUNK_DMAS):
            cp = pltpu.make_async_copy(
                table_hbm.at[idx_v.at[c * CHUNK_DMAS + j]],
                buf.at[pl.ds(j * IDX_MINOR, IDX_MINOR)],
                sem,
            )
            cp.start()
            cps.append(cp)
        return cps

    pending = fire(0)
    for c in range(N_CHUNKS):
        buf = bufs[c % 2]
        drain = pending
        if c + 1 < N_CHUNKS:
            pending = fire(c + 1)
        for cp in drain:
            cp.wait()

        lanes = lax.iota(jnp.int32, 16)

        def voxel_body(g, _, buf=buf, c=c, lanes=lanes):
            for u in range(4):
                v = g * 4 + u
                base = v * HIST
                acc = _tree_sum([buf[base + l] for l in range(HIST)])
                ssv = acc * acc
                for k in (1, 2, 4, 8):
                    ssv = ssv + _lane_gather(ssv, lanes ^ k)
                bits = lax.bitcast_convert_type(ssv, jnp.int32)
                bits = jnp.int32(0x5F3759DF) - (bits >> 1)
                y = lax.bitcast_convert_type(bits, jnp.float32)
                half = ssv * 0.5
                for _i in range(3):
                    y = y * (1.5 - half * y * y)
                out_v[c * CHUNK_V + v] = acc * y
            return 0

        lax.fori_loop(0, CHUNK_V // 4, voxel_body, 0)

    pltpu.sync_copy(out_v, out_hbm.at[pl.ds(wid * VPW, VPW)])


def kernel(segment_lists, weight):
    idx3 = segment_lists.astype(jnp.int32).reshape(NW, IDX_ROWS, IDX_MINOR)
    return _sc_encode(idx3, weight)


# speedup vs baseline: 1.2063x; 1.2063x over previous
import functools

import jax
import jax.numpy as jnp
from jax import lax
from jax.experimental import pallas as pl
from jax.experimental.pallas import tpu as pltpu
from jax.experimental.pallas import tpu_sc as plsc

BATCH = 16384
HIST = 20
EMBED_DIM = 16
NC = 2
NS = 16
NW = NC * NS
VPW = BATCH // NW
CHUNK_V = 128
CHUNK_ROWS = CHUNK_V * HIST
N_CHUNKS = VPW // CHUNK_V

_mesh = plsc.VectorSubcoreMesh(core_axis_name="c", subcore_axis_name="s")

_GATHER_DNUMS = lax.GatherDimensionNumbers(
    offset_dims=(), collapsed_slice_dims=(0,), start_index_map=(0,))


def _lane_gather(x, idx):
    return lax.gather(
        x, idx[:, None], _GATHER_DNUMS, (1,),
        mode=lax.GatherScatterMode.PROMISE_IN_BOUNDS)


def _tree_sum(vals):
    while len(vals) > 1:
        nxt = [a + b for a, b in zip(vals[::2], vals[1::2])]
        if len(vals) % 2:
            nxt.append(vals[-1])
        vals = nxt
    return vals[0]


@functools.partial(
    pl.kernel,
    out_type=jax.ShapeDtypeStruct((BATCH, EMBED_DIM), jnp.float32),
    mesh=_mesh,
    scratch_types=[
        pltpu.VMEM((HIST, VPW), jnp.int32),
        pltpu.VMEM((CHUNK_ROWS, EMBED_DIM), jnp.float32),
        pltpu.VMEM((CHUNK_ROWS, EMBED_DIM), jnp.float32),
        pltpu.VMEM((VPW, EMBED_DIM), jnp.float32),
        pltpu.SemaphoreType.DMA,
        pltpu.SemaphoreType.DMA,
    ],
    compiler_params=pltpu.CompilerParams(use_tc_tiling_on_sc=False),
)
def _sc_encode(segt_hbm, table_hbm, out_hbm, idx_v, rows_a, rows_b, out_v,
               sem_a, sem_b):
    wid = lax.axis_index("s") * NC + lax.axis_index("c")
    pltpu.sync_copy(segt_hbm.at[:, pl.ds(wid * VPW, VPW)], idx_v)

    bufs = (rows_a, rows_b)
    sems = (sem_a, sem_b)

    def fire(c):
        buf, sem = bufs[c % 2], sems[c % 2]
        cps = []
        for l in range(HIST):
            cp = pltpu.make_async_copy(
                table_hbm.at[idx_v.at[l, pl.ds(c * CHUNK_V, CHUNK_V)]],
                buf.at[pl.ds(l * CHUNK_V, CHUNK_V)],
                sem,
            )
            cp.start()
            cps.append(cp)
        return cps

    pending = fire(0)
    for c in range(N_CHUNKS):
        buf = bufs[c % 2]
        drain = pending
        if c + 1 < N_CHUNKS:
            pending = fire(c + 1)
        for cp in drain:
            cp.wait()

        lanes = lax.iota(jnp.int32, 16)

        def voxel_body(v, _, buf=buf, c=c, lanes=lanes):
            acc = _tree_sum([buf[l * CHUNK_V + v] for l in range(HIST)])
            ssv = acc * acc
            for k in (1, 2, 4, 8):
                ssv = ssv + _lane_gather(ssv, lanes ^ k)
            bits = lax.bitcast_convert_type(ssv, jnp.int32)
            bits = jnp.int32(0x5F3759DF) - (bits >> 1)
            y = lax.bitcast_convert_type(bits, jnp.float32)
            half = ssv * 0.5
            for _i in range(3):
                y = y * (1.5 - half * y * y)
            out_v[c * CHUNK_V + v] = acc * y
            return 0

        lax.fori_loop(0, CHUNK_V, voxel_body, 0)

    pltpu.sync_copy(out_v, out_hbm.at[pl.ds(wid * VPW, VPW)])


def kernel(segment_lists, weight):
    seg_t = segment_lists.astype(jnp.int32).T
    return _sc_encode(seg_t, weight)
